# Initial kernel scaffold; baseline (speedup 1.0000x reference)
import jax
import jax.numpy as jnp
from jax.experimental import pallas as pl


def kernel(x, edge_index, graph_ids, W1, b1, gamma1, beta1, W2, b2, gamma2, beta2, fc1_W, fc1_b, fc2_W, fc2_b):
    raise NotImplementedError("write your pallas kernel here")


# speedup vs baseline: 5.2535x; 5.2535x over previous
import functools

import jax
import jax.numpy as jnp
from jax import lax
from jax.experimental import pallas as pl
from jax.experimental.pallas import tpu as pltpu
from jax.experimental.pallas import tpu_sc as plsc

N = 10000
E = 320000
D = 128
C = 256
G = 128
H = 512
OUT = 128

NC = 2
NS = 16
L = 16
CH = C // 2

N_PAD = 10240

_sc_mesh = plsc.VectorSubcoreMesh(
    core_axis_name="c", subcore_axis_name="s", num_cores=NC, num_subcores=NS)

_i32 = jnp.int32
_f32 = jnp.float32


_DCH = 80
_DEPW = E // NS
_DNCH = _DEPW // _DCH
_DZB = N_PAD // NS


@functools.partial(
    pl.kernel,
    out_type=(jax.ShapeDtypeStruct((N_PAD,), _f32),
              jax.ShapeDtypeStruct((N_PAD,), _f32)),
    mesh=_sc_mesh,
    scratch_types=[
        pltpu.VMEM((_DCH,), _i32),
        pltpu.VMEM((_DCH,), _f32),
        pltpu.VMEM((_DZB,), _f32),
        pltpu.VMEM_SHARED((N_PAD,), _f32),
    ],
)
def _deg_kernel(src_hbm, dst_hbm, dout_hbm, din_hbm, idx_v, ones_v, zb_v,
                deg_sh):
    c = lax.axis_index("c")
    s = lax.axis_index("s")
    one16 = jnp.ones((L,), _f32)
    zero16 = jnp.zeros((L,), _f32)
    for j in range(_DCH // L):
        ones_v[pl.ds(j * L, L)] = one16

    def _zb(j, carry):
        zb_v[pl.ds(j * L, L)] = zero16
        return carry
    lax.fori_loop(0, _DZB // L, _zb, 0)
    pltpu.sync_copy(zb_v, deg_sh.at[pl.ds(s * _DZB, _DZB)])
    plsc.subcore_barrier()

    def _step(i, carry):
        base = s * _DEPW + i * _DCH

        @pl.when(c == 0)
        def _():
            pltpu.sync_copy(src_hbm.at[pl.ds(base, _DCH)], idx_v)

        @pl.when(c == 1)
        def _():
            pltpu.sync_copy(dst_hbm.at[pl.ds(base, _DCH)], idx_v)

        pltpu.sync_copy(ones_v, deg_sh.at[idx_v], add=True)
        return carry
    lax.fori_loop(0, _DNCH, _step, 0)
    plsc.subcore_barrier()

    pltpu.sync_copy(deg_sh.at[pl.ds(s * _DZB, _DZB)], zb_v)

    @pl.when(c == 0)
    def _():
        pltpu.sync_copy(zb_v, dout_hbm.at[pl.ds(s * _DZB, _DZB)])

    @pl.when(c == 1)
    def _():
        pltpu.sync_copy(zb_v, din_hbm.at[pl.ds(s * _DZB, _DZB)])


_ECH = 80
_EEPW = E // NS
_ENCH = _EEPW // _ECH
_ERPW = 640
_EZR = 80


@functools.partial(
    pl.kernel,
    out_type=(jax.ShapeDtypeStruct((N, CH), _f32),
              jax.ShapeDtypeStruct((N, CH), _f32)),
    mesh=_sc_mesh,
    scratch_types=[
        pltpu.VMEM((_ECH,), _i32),
        pltpu.VMEM((_ECH,), _i32),
        pltpu.VMEM((_ECH, CH), _f32),
        pltpu.VMEM((_EZR, CH), _f32),
        pltpu.VMEM_SHARED((N, CH), _f32),
        pltpu.SemaphoreType.DMA,
    ],
)
def _edge_kernel(hlo_hbm, hhi_hbm, src_hbm, dst_hbm, alo_hbm, ahi_hbm,
                 sidx_v, didx_v, rows_v, zb_v, acc_sh, sem):
    c = lax.axis_index("c")
    s = lax.axis_index("s")
    iota16 = jnp.arange(L, dtype=_i32)
    zero16 = jnp.zeros((L,), _f32)

    def _zrow(r, carry):
        for j in range(CH // L):
            zb_v[r, pl.ds(j * L, L)] = zero16
        return carry
    lax.fori_loop(0, _EZR, _zrow, 0)
    ncp = jnp.where(s < NS - 1, _ERPW // _EZR,
                    (N - (NS - 1) * _ERPW) // _EZR)

    def _zcp(k, carry):
        pltpu.sync_copy(zb_v, acc_sh.at[pl.ds(s * _ERPW + k * _EZR, _EZR)])
        return carry
    lax.fori_loop(0, ncp, _zcp, 0)
    plsc.subcore_barrier()

    def _step(i, carry):
        base = s * _EEPW + i * _ECH
        pltpu.sync_copy(src_hbm.at[pl.ds(base, _ECH)], sidx_v)
        pltpu.sync_copy(dst_hbm.at[pl.ds(base, _ECH)], didx_v)

        @pl.when(c == 0)
        def _():
            pltpu.async_copy(hlo_hbm.at[sidx_v], rows_v, sem).wait()

        @pl.when(c == 1)
        def _():
            pltpu.async_copy(hhi_hbm.at[sidx_v], rows_v, sem).wait()

        pltpu.sync_copy(rows_v, acc_sh.at[didx_v], add=True)
        return carry
    lax.fori_loop(0, _ENCH, _step, 0)
    plsc.subcore_barrier()

    def _wb(k, carry):
        off = s * _ERPW + k * _EZR
        pltpu.sync_copy(acc_sh.at[pl.ds(off, _EZR)], zb_v)

        @pl.when(c == 0)
        def _():
            pltpu.sync_copy(zb_v, alo_hbm.at[pl.ds(off, _EZR)])

        @pl.when(c == 1)
        def _():
            pltpu.sync_copy(zb_v, ahi_hbm.at[pl.ds(off, _EZR)])
        return carry
    lax.fori_loop(0, ncp, _wb, 0)


_SNPW = 640
_SRCH = 80
_SGPW = G // NS


@functools.partial(
    pl.kernel,
    out_type=(jax.ShapeDtypeStruct((G, CH), _f32),
              jax.ShapeDtypeStruct((G, CH), _f32)),
    mesh=_sc_mesh,
    scratch_types=[
        pltpu.VMEM((_SNPW,), _i32),
        pltpu.VMEM((_SRCH, CH), _f32),
        pltpu.VMEM((G, CH), _f32),
        pltpu.VMEM((_SGPW, CH), _f32),
        pltpu.VMEM((_SGPW, CH), _f32),
        pltpu.VMEM_SHARED((NS, G, CH), _f32),
    ],
)
def _segmax_kernel(flo_hbm, fhi_hbm, gid_hbm, plo_hbm, phi_hbm,
                   gid_v, rows_v, acc_v, tbuf_v, obuf_v, pool_sh):
    c = lax.axis_index("c")
    s = lax.axis_index("s")
    iota16 = jnp.arange(L, dtype=_i32)
    zero16 = jnp.zeros((L,), _f32)

    def _zrow(r, carry):
        for j in range(CH // L):
            acc_v[r, pl.ds(j * L, L)] = zero16
        return carry
    lax.fori_loop(0, G, _zrow, 0)

    pltpu.sync_copy(gid_hbm.at[pl.ds(s * _SNPW, _SNPW)], gid_v)
    nchunks = jnp.where(s < NS - 1, _SNPW // _SRCH, (N - (NS - 1) * _SNPW) // _SRCH)

    def _chunk(k, carry):
        @pl.when(c == 0)
        def _():
            pltpu.sync_copy(flo_hbm.at[pl.ds(s * _SNPW + k * _SRCH, _SRCH)],
                            rows_v)

        @pl.when(c == 1)
        def _():
            pltpu.sync_copy(fhi_hbm.at[pl.ds(s * _SNPW + k * _SRCH, _SRCH)],
                            rows_v)

        def _grp(g2, carry2):
            gvec = gid_v[pl.ds(k * _SRCH + g2 * L, L)]
            for lane in range(L):
                r = g2 * L + lane
                gid = gvec[lane]
                for j in range(CH // L):
                    cs = pl.ds(j * L, L)
                    vals = rows_v[r, cs]
                    cur = acc_v[gid, cs]
                    acc_v[gid, cs] = jnp.maximum(cur, vals)
            return carry2
        lax.fori_loop(0, _SRCH // L, _grp, 0)
        return carry
    lax.fori_loop(0, nchunks, _chunk, 0)

    pltpu.sync_copy(acc_v, pool_sh.at[s])
    plsc.subcore_barrier()

    gbase = s * _SGPW
    for r in range(_SGPW):
        for j in range(CH // L):
            obuf_v[r, pl.ds(j * L, L)] = zero16

    def _comb(p, carry):
        pltpu.sync_copy(pool_sh.at[p, pl.ds(gbase, _SGPW)], tbuf_v)
        for r in range(_SGPW):
            for j in range(CH // L):
                cur = obuf_v[r, pl.ds(j * L, L)]
                val = tbuf_v[r, pl.ds(j * L, L)]
                obuf_v[r, pl.ds(j * L, L)] = jnp.maximum(cur, val)
        return carry
    lax.fori_loop(0, NS, _comb, 0)

    @pl.when(c == 0)
    def _():
        pltpu.sync_copy(obuf_v, plo_hbm.at[pl.ds(gbase, _SGPW)])

    @pl.when(c == 1)
    def _():
        pltpu.sync_copy(obuf_v, phi_hbm.at[pl.ds(gbase, _SGPW)])


_BLK = 1000
_NBLK = N // _BLK


def _norm_from_deg(deg):
    return jnp.where(deg > 0.0, lax.rsqrt(jnp.maximum(deg, 1.0)), 0.0)


def _mm1_body(x_ref, w_ref, dout_ref, lo_ref, hi_ref):
    h = jnp.dot(x_ref[...], w_ref[...], preferred_element_type=_f32)
    h = h * _norm_from_deg(dout_ref[...])
    lo_ref[...] = h[:, :CH]
    hi_ref[...] = h[:, CH:]


def _mm1(x, W1, deg_out):
    return pl.pallas_call(
        _mm1_body,
        grid=(_NBLK,),
        in_specs=[
            pl.BlockSpec((_BLK, D), lambda i: (i, 0)),
            pl.BlockSpec((D, C), lambda i: (0, 0)),
            pl.BlockSpec((_BLK, 1), lambda i: (i, 0)),
        ],
        out_specs=[
            pl.BlockSpec((_BLK, CH), lambda i: (i, 0)),
            pl.BlockSpec((_BLK, CH), lambda i: (i, 0)),
        ],
        out_shape=[jax.ShapeDtypeStruct((N, CH), _f32),
                   jax.ShapeDtypeStruct((N, CH), _f32)],
        compiler_params=pltpu.CompilerParams(
            dimension_semantics=("arbitrary",)),
    )(x, W1, deg_out)


def _pre_bn(alo, ahi, din, b):
    t = jnp.concatenate([alo, ahi], axis=1)
    return t * _norm_from_deg(din) + b


def _stats_body(alo_ref, ahi_ref, din_ref, b_ref, out_ref):
    i = pl.program_id(0)
    t = _pre_bn(alo_ref[...], ahi_ref[...], din_ref[...], b_ref[...])
    ps = jnp.sum(t, axis=0, keepdims=True)
    pq = jnp.sum(t * t, axis=0, keepdims=True)
    part = jnp.concatenate([ps, pq], axis=0)

    @pl.when(i == 0)
    def _():
        out_ref[...] = part

    @pl.when(i > 0)
    def _():
        out_ref[...] = out_ref[...] + part

    @pl.when(i == _NBLK - 1)
    def _():
        acc = out_ref[...]
        mu = acc[0:1] / N
        var = acc[1:2] / N - mu * mu
        out_ref[...] = jnp.concatenate(
            [mu, lax.rsqrt(var + 1e-5)], axis=0)


def _bn_stats(alo, ahi, din, b):
    return pl.pallas_call(
        _stats_body,
        grid=(_NBLK,),
        in_specs=[
            pl.BlockSpec((_BLK, CH), lambda i: (i, 0)),
            pl.BlockSpec((_BLK, CH), lambda i: (i, 0)),
            pl.BlockSpec((_BLK, 1), lambda i: (i, 0)),
            pl.BlockSpec((1, C), lambda i: (0, 0)),
        ],
        out_specs=pl.BlockSpec((2, C), lambda i: (0, 0)),
        out_shape=jax.ShapeDtypeStruct((2, C), _f32),
        compiler_params=pltpu.CompilerParams(
            dimension_semantics=("arbitrary",)),
    )(alo, ahi, din, b)


def _bn_relu(alo_ref, ahi_ref, din_ref, b_ref, g_ref, bt_ref, st_ref):
    t = _pre_bn(alo_ref[...], ahi_ref[...], din_ref[...], b_ref[...])
    mu = st_ref[0:1]
    inv = st_ref[1:2]
    return jnp.maximum(g_ref[...] * (t - mu) * inv + bt_ref[...], 0.0)


def _bn_mm2_body(alo_ref, ahi_ref, din_ref, b_ref, g_ref, bt_ref, st_ref,
                 w_ref, dout_ref, lo_ref, hi_ref):
    hn = _bn_relu(alo_ref, ahi_ref, din_ref, b_ref, g_ref, bt_ref, st_ref)
    h = jnp.dot(hn, w_ref[...], preferred_element_type=_f32)
    h = h * _norm_from_deg(dout_ref[...])
    lo_ref[...] = h[:, :CH]
    hi_ref[...] = h[:, CH:]


def _bn_mm2(alo, ahi, din, b, gamma, beta, stats, W2, deg_out):
    return pl.pallas_call(
        _bn_mm2_body,
        grid=(_NBLK,),
        in_specs=[
            pl.BlockSpec((_BLK, CH), lambda i: (i, 0)),
            pl.BlockSpec((_BLK, CH), lambda i: (i, 0)),
            pl.BlockSpec((_BLK, 1), lambda i: (i, 0)),
            pl.BlockSpec((1, C), lambda i: (0, 0)),
            pl.BlockSpec((1, C), lambda i: (0, 0)),
            pl.BlockSpec((1, C), lambda i: (0, 0)),
            pl.BlockSpec((2, C), lambda i: (0, 0)),
            pl.BlockSpec((C, C), lambda i: (0, 0)),
            pl.BlockSpec((_BLK, 1), lambda i: (i, 0)),
        ],
        out_specs=[
            pl.BlockSpec((_BLK, CH), lambda i: (i, 0)),
            pl.BlockSpec((_BLK, CH), lambda i: (i, 0)),
        ],
        out_shape=[jax.ShapeDtypeStruct((N, CH), _f32),
                   jax.ShapeDtypeStruct((N, CH), _f32)],
        compiler_params=pltpu.CompilerParams(
            dimension_semantics=("arbitrary",)),
    )(alo, ahi, din, b, gamma, beta, stats, W2, deg_out)


def _bn_out_body(alo_ref, ahi_ref, din_ref, b_ref, g_ref, bt_ref, st_ref,
                 lo_ref, hi_ref):
    hn = _bn_relu(alo_ref, ahi_ref, din_ref, b_ref, g_ref, bt_ref, st_ref)
    lo_ref[...] = hn[:, :CH]
    hi_ref[...] = hn[:, CH:]


def _bn_out(alo, ahi, din, b, gamma, beta, stats):
    return pl.pallas_call(
        _bn_out_body,
        grid=(_NBLK,),
        in_specs=[
            pl.BlockSpec((_BLK, CH), lambda i: (i, 0)),
            pl.BlockSpec((_BLK, CH), lambda i: (i, 0)),
            pl.BlockSpec((_BLK, 1), lambda i: (i, 0)),
            pl.BlockSpec((1, C), lambda i: (0, 0)),
            pl.BlockSpec((1, C), lambda i: (0, 0)),
            pl.BlockSpec((1, C), lambda i: (0, 0)),
            pl.BlockSpec((2, C), lambda i: (0, 0)),
        ],
        out_specs=[
            pl.BlockSpec((_BLK, CH), lambda i: (i, 0)),
            pl.BlockSpec((_BLK, CH), lambda i: (i, 0)),
        ],
        out_shape=[jax.ShapeDtypeStruct((N, CH), _f32),
                   jax.ShapeDtypeStruct((N, CH), _f32)],
        compiler_params=pltpu.CompilerParams(
            dimension_semantics=("arbitrary",)),
    )(alo, ahi, din, b, gamma, beta, stats)


def _head_body(plo_ref, phi_ref, w1_ref, b1_ref, w2_ref, b2_ref, out_ref):
    pooled = jnp.concatenate([plo_ref[...], phi_ref[...]], axis=1)
    z = jnp.maximum(
        jnp.dot(pooled, w1_ref[...], preferred_element_type=_f32) + b1_ref[...],
        0.0)
    out_ref[...] = (jnp.dot(z, w2_ref[...], preferred_element_type=_f32)
                    + b2_ref[...])


def _head(plo, phi, fc1_W, fc1_b, fc2_W, fc2_b):
    return pl.pallas_call(
        _head_body,
        out_shape=jax.ShapeDtypeStruct((G, OUT), _f32),
    )(plo, phi, fc1_W, fc1_b, fc2_W, fc2_b)


def kernel(x, edge_index, graph_ids, W1, b1, gamma1, beta1, W2, b2, gamma2,
           beta2, fc1_W, fc1_b, fc2_W, fc2_b):
    src = edge_index[0]
    dst = edge_index[1]

    dout_p, din_p = _deg_kernel(src, dst)
    deg_out = dout_p[:N].reshape(N, 1)
    deg_in = din_p[:N].reshape(N, 1)

    b1r = b1.reshape(1, C)
    g1r = gamma1.reshape(1, C)
    bt1r = beta1.reshape(1, C)
    b2r = b2.reshape(1, C)
    g2r = gamma2.reshape(1, C)
    bt2r = beta2.reshape(1, C)

    h_lo, h_hi = _mm1(x, W1, deg_out)
    a_lo, a_hi = _edge_kernel(h_lo, h_hi, src, dst)
    st1 = _bn_stats(a_lo, a_hi, deg_in, b1r)
    h2_lo, h2_hi = _bn_mm2(a_lo, a_hi, deg_in, b1r, g1r, bt1r, st1, W2,
                           deg_out)
    a2_lo, a2_hi = _edge_kernel(h2_lo, h2_hi, src, dst)
    st2 = _bn_stats(a2_lo, a2_hi, deg_in, b2r)
    f_lo, f_hi = _bn_out(a2_lo, a2_hi, deg_in, b2r, g2r, bt2r, st2)

    gid_pad = jnp.pad(graph_ids, (0, N_PAD - N), constant_values=G - 1)
    p_lo, p_hi = _segmax_kernel(f_lo, f_hi, gid_pad)

    return _head(p_lo, p_hi, fc1_W, fc1_b.reshape(1, H), fc2_W,
                 fc2_b.reshape(1, OUT))
